# baseline (speedup 1.0000x reference)
import functools

import jax
import jax.numpy as jnp
from jax import lax
from jax.experimental import pallas as pl
from jax.experimental.pallas import tpu as pltpu
from jax.experimental.pallas import tpu_sc as plsc

B = 16384
N_REG = 1000000
N_EMB = 20
NC = 2
NS = 16
NW = NC * NS
BPW = B // NW
CHUNK = 128
NCHUNK = BPW // CHUNK
L = 16


def _sc_body(tabf, bias_ref, idx_hbm, c_hbm, wb_hbm, out_hbm,
             idx_v, eidx_v, cols_v, c_v, bias_v, out_v, wb_v, sem):
    wid = lax.axis_index("s") * NC + lax.axis_index("c")

    pltpu.sync_copy(idx_hbm.at[wid], idx_v)
    pltpu.sync_copy(c_hbm.at[wid], c_v)
    pltpu.sync_copy(wb_hbm, wb_v)

    for k in range(NCHUNK):
        def mkidx(t, _, k=k):
            r = idx_v[k, pl.ds(t * L, L)]
            for e in range(N_EMB):
                eidx_v[k, e, pl.ds(t * L, L)] = r + e * N_REG
            return _
        lax.fori_loop(0, CHUNK // L, mkidx, 0)

    copies = []
    for k in range(NCHUNK):
        for e in range(N_EMB):
            copies.append(pltpu.async_copy(
                tabf.at[eidx_v.at[k, e]],
                cols_v.at[e, pl.ds(k * CHUNK, CHUNK)], sem))
        copies.append(pltpu.async_copy(
            bias_ref.at[idx_v.at[k]], bias_v.at[pl.ds(k * CHUNK, CHUNK)],
            sem))
    for c in copies:
        c.wait()

    wbv = [wb_v[pl.ds(16 * j, 16)] for j in range(3)]
    wbs = [wbv[j // 16][j % 16] for j in range(2 * N_EMB)]
    w0s = wbs[:N_EMB]
    b0s = wbs[N_EMB:]

    def group(t, _):
        c = c_v[pl.ds(t * L, L)]
        acc = bias_v[pl.ds(t * L, L)]
        for e in range(N_EMB):
            w_e = cols_v[e, pl.ds(t * L, L)]
            h_e = jnp.maximum(c * w0s[e] + b0s[e], 0.0)
            acc = acc + h_e * w_e
        out_v[pl.ds(t * L, L)] = acc
        return _
    lax.fori_loop(0, BPW // L, group, 0)

    pltpu.sync_copy(out_v, out_hbm.at[wid])


@jax.jit
def _run(tabf, bias, idx, coords, wb):
    mesh = plsc.VectorSubcoreMesh(core_axis_name="c", subcore_axis_name="s")
    f = functools.partial(
        pl.kernel,
        mesh=mesh,
        out_type=jax.ShapeDtypeStruct((NW, BPW), jnp.float32),
        scratch_types=[
            pltpu.VMEM((NCHUNK, CHUNK), jnp.int32),
            pltpu.VMEM((NCHUNK, N_EMB, CHUNK), jnp.int32),
            pltpu.VMEM((N_EMB, BPW), jnp.float32),
            pltpu.VMEM((BPW,), jnp.float32),
            pltpu.VMEM((BPW,), jnp.float32),
            pltpu.VMEM((BPW,), jnp.float32),
            pltpu.VMEM((48,), jnp.float32),
            pltpu.SemaphoreType.DMA,
        ],
        compiler_params=pltpu.CompilerParams(
            needs_layout_passes=False, use_tc_tiling_on_sc=False),
    )(_sc_body)
    return f(tabf, bias, idx, coords, wb)


def kernel(coordinates, region_ix, W0, b0, weight1_table, bias1_table):
    tabf = jnp.concatenate(
        [weight1_table[:, e, 0] for e in range(N_EMB)])
    bias = bias1_table.reshape(-1)
    idx = region_ix.astype(jnp.int32).reshape(NW, NCHUNK, CHUNK)
    coords = coordinates.reshape(NW, BPW)
    wb = jnp.concatenate(
        [W0.reshape(-1) / 20000.0, b0, jnp.zeros((8,), jnp.float32)])
    out = _run(tabf, bias, idx, coords, wb)
    return out.reshape(B, 1)


# speedup vs baseline: 1.1689x; 1.1689x over previous
import functools

import jax
import jax.numpy as jnp
from jax import lax
from jax.experimental import pallas as pl
from jax.experimental.pallas import tpu as pltpu
from jax.experimental.pallas import tpu_sc as plsc

B = 16384
N_REG = 1000000
N_EMB = 20
NC = 2
NS = 16
NW = NC * NS
BPW = B // NW
CHUNK = 128
NCHUNK = BPW // CHUNK
L = 16


def _sc_body(tabf, bias_ref, idx_hbm, c_hbm, wb_hbm, out_hbm,
             idx_v, eidx_v, cols_v, c_v, bias_v, out_v, wb_v, sem):
    wid = lax.axis_index("s") * NC + lax.axis_index("c")

    pltpu.sync_copy(idx_hbm.at[wid], idx_v)
    pltpu.sync_copy(c_hbm.at[wid], c_v)
    pltpu.sync_copy(wb_hbm, wb_v)

    for k in range(NCHUNK):
        def mkidx(t, _, k=k):
            r = idx_v[k, pl.ds(t * L, L)] * N_EMB
            for e in range(N_EMB):
                eidx_v[k, e, pl.ds(t * L, L)] = r + e
            return _
        lax.fori_loop(0, CHUNK // L, mkidx, 0)

    copies = []
    for k in range(NCHUNK):
        for e in range(N_EMB):
            copies.append(pltpu.async_copy(
                tabf.at[eidx_v.at[k, e]],
                cols_v.at[e, pl.ds(k * CHUNK, CHUNK)], sem))
        copies.append(pltpu.async_copy(
            bias_ref.at[idx_v.at[k]], bias_v.at[pl.ds(k * CHUNK, CHUNK)],
            sem))
    for c in copies:
        c.wait()

    wbv = [wb_v[pl.ds(16 * j, 16)] for j in range(3)]
    wbs = [wbv[j // 16][j % 16] for j in range(2 * N_EMB)]
    w0s = wbs[:N_EMB]
    b0s = wbs[N_EMB:]

    def group(t, _):
        c = c_v[pl.ds(t * L, L)]
        acc = bias_v[pl.ds(t * L, L)]
        for e in range(N_EMB):
            w_e = cols_v[e, pl.ds(t * L, L)]
            h_e = jnp.maximum(c * w0s[e] + b0s[e], 0.0)
            acc = acc + h_e * w_e
        out_v[pl.ds(t * L, L)] = acc
        return _
    lax.fori_loop(0, BPW // L, group, 0)

    pltpu.sync_copy(out_v, out_hbm.at[wid])


@jax.jit
def _run(tabf, bias, idx, coords, wb):
    mesh = plsc.VectorSubcoreMesh(core_axis_name="c", subcore_axis_name="s")
    f = functools.partial(
        pl.kernel,
        mesh=mesh,
        out_type=jax.ShapeDtypeStruct((NW, BPW), jnp.float32),
        scratch_types=[
            pltpu.VMEM((NCHUNK, CHUNK), jnp.int32),
            pltpu.VMEM((NCHUNK, N_EMB, CHUNK), jnp.int32),
            pltpu.VMEM((N_EMB, BPW), jnp.float32),
            pltpu.VMEM((BPW,), jnp.float32),
            pltpu.VMEM((BPW,), jnp.float32),
            pltpu.VMEM((BPW,), jnp.float32),
            pltpu.VMEM((48,), jnp.float32),
            pltpu.SemaphoreType.DMA,
        ],
        compiler_params=pltpu.CompilerParams(
            needs_layout_passes=False, use_tc_tiling_on_sc=False),
    )(_sc_body)
    return f(tabf, bias, idx, coords, wb)


def kernel(coordinates, region_ix, W0, b0, weight1_table, bias1_table):
    tabf = weight1_table.reshape(N_EMB * N_REG)
    bias = bias1_table.reshape(-1)
    idx = region_ix.astype(jnp.int32).reshape(NW, NCHUNK, CHUNK)
    coords = coordinates.reshape(NW, BPW)
    wb = jnp.concatenate(
        [W0.reshape(-1) / 20000.0, b0, jnp.zeros((8,), jnp.float32)])
    out = _run(tabf, bias, idx, coords, wb)
    return out.reshape(B, 1)
